# baseline (speedup 1.0000x reference)
import functools

import jax
import jax.numpy as jnp
from jax.experimental import pallas as pl
from jax.experimental.pallas import tpu as pltpu

_THRESH = 0.35667494393873245
_N_MIN = 16 * 512 * 512 // 16


def _ce_loss(logits_ref, labels_ref):
    x = logits_ref[0]
    lab = labels_ref[0]
    m = jnp.max(x, axis=0)
    s = jnp.sum(jnp.exp(x - m[None, :, :]), axis=0)
    lse = m + jnp.log(s)
    c = jax.lax.broadcasted_iota(jnp.int32, x.shape, 0)
    lg = jnp.sum(jnp.where(lab[None, :, :] == c, x, 0.0), axis=0)
    return lse - lg


def _ce_stats_block(logits_ref, labels_ref, out_ref, acc_ref):
    i = pl.program_id(0) * pl.num_programs(1) + pl.program_id(1)

    @pl.when(i == 0)
    def _init():
        acc_ref[...] = jnp.zeros_like(acc_ref)

    l = _ce_loss(logits_ref, labels_ref)
    keep = l > jnp.float32(_THRESH)
    acc_ref[0] += jnp.where(keep, l, 0.0)
    acc_ref[1] += keep.astype(jnp.float32)

    @pl.when(i == pl.num_programs(0) * pl.num_programs(1) - 1)
    def _fin():
        s_keep = jnp.sum(acc_ref[0])
        c_keep = jnp.sum(acc_ref[1])
        out_ref[0] = s_keep
        out_ref[1] = c_keep
        out_ref[2] = s_keep / c_keep


def _ce_block(logits_ref, labels_ref, loss_ref):
    loss_ref[0] = _ce_loss(logits_ref, labels_ref)


def _select_block(loss_ref, out_ref):
    v = loss_ref[...]
    p = jax.lax.bitcast_convert_type(v, jnp.int32)
    k = jnp.int32(_N_MIN)

    def body(i, t):
        cand = t + jax.lax.shift_left(jnp.int32(1), 30 - i)
        cnt = jnp.sum((p >= cand).astype(jnp.int32))
        return jnp.where(cnt >= k, cand, t)

    t = jax.lax.fori_loop(0, 31, body, jnp.int32(0))
    tv = jax.lax.bitcast_convert_type(t, jnp.float32)

    gt = p > t
    cnt_gt = jnp.sum(gt.astype(jnp.int32))
    sum_gt = jnp.sum(jnp.where(gt, v, 0.0))
    out_ref[0, 0] = (sum_gt + (k - cnt_gt).astype(jnp.float32) * tv) / jnp.float32(_N_MIN)


def _topk_branch(logits, labels):
    N, C, H, W = logits.shape
    BH = 256
    loss = pl.pallas_call(
        _ce_block,
        grid=(N, H // BH),
        in_specs=[
            pl.BlockSpec((1, C, BH, W), lambda n, h: (n, 0, h, 0)),
            pl.BlockSpec((1, BH, W), lambda n, h: (n, h, 0)),
        ],
        out_specs=pl.BlockSpec((1, BH, W), lambda n, h: (n, h, 0)),
        out_shape=jax.ShapeDtypeStruct((N, H, W), jnp.float32),
    )(logits, labels)
    flat = loss.reshape(N * H * W // 1024, 1024)
    out = pl.pallas_call(
        _select_block,
        in_specs=[pl.BlockSpec(flat.shape, lambda: (0, 0))],
        out_specs=pl.BlockSpec(memory_space=pltpu.SMEM),
        out_shape=jax.ShapeDtypeStruct((1, 1), jnp.float32),
    )(flat)
    return out[0, 0]


@jax.jit
def kernel(logits, labels):
    N, C, H, W = logits.shape
    labels = labels.astype(jnp.int32)
    BH = 256
    stats = pl.pallas_call(
        _ce_stats_block,
        grid=(N, H // BH),
        in_specs=[
            pl.BlockSpec((1, C, BH, W), lambda n, h: (n, 0, h, 0)),
            pl.BlockSpec((1, BH, W), lambda n, h: (n, h, 0)),
        ],
        out_specs=pl.BlockSpec(memory_space=pltpu.SMEM),
        out_shape=jax.ShapeDtypeStruct((3,), jnp.float32),
        scratch_shapes=[pltpu.VMEM((2, BH, W), jnp.float32)],
    )(logits, labels)

    return jax.lax.cond(
        stats[1] >= jnp.float32(_N_MIN),
        lambda: stats[2],
        lambda: _topk_branch(logits, labels),
    )


# speedup vs baseline: 1.0197x; 1.0197x over previous
import functools

import jax
import jax.numpy as jnp
from jax.experimental import pallas as pl
from jax.experimental.pallas import tpu as pltpu

_THRESH = 0.35667494393873245
_N_MIN = 16 * 512 * 512 // 16


def _ce_loss(logits_ref, labels_ref):
    x = logits_ref[0]
    lab = labels_ref[0]
    m = jnp.max(x, axis=0)
    s = jnp.sum(jnp.exp(x - m[None, :, :]), axis=0)
    lse = m + jnp.log(s)
    c = jax.lax.broadcasted_iota(jnp.int32, x.shape, 0)
    lg = jnp.sum(jnp.where(lab[None, :, :] == c, x, 0.0), axis=0)
    return lse - lg


def _ce_stats_block(logits_ref, labels_ref, out_ref, acc_ref):
    i = pl.program_id(0) * pl.num_programs(1) + pl.program_id(1)

    @pl.when(i == 0)
    def _init():
        acc_ref[...] = jnp.zeros_like(acc_ref)

    l = _ce_loss(logits_ref, labels_ref)
    keep = l > jnp.float32(_THRESH)
    acc_ref[0] += jnp.where(keep, l, 0.0)
    acc_ref[1] += keep.astype(jnp.float32)

    @pl.when(i == pl.num_programs(0) * pl.num_programs(1) - 1)
    def _fin():
        s_keep = jnp.sum(acc_ref[0])
        c_keep = jnp.sum(acc_ref[1])
        out_ref[0] = s_keep
        out_ref[1] = c_keep
        out_ref[2] = s_keep / c_keep


def _ce_block(logits_ref, labels_ref, loss_ref):
    loss_ref[0] = _ce_loss(logits_ref, labels_ref)


def _select_block(loss_ref, out_ref):
    v = loss_ref[...]
    p = jax.lax.bitcast_convert_type(v, jnp.int32)
    k = jnp.int32(_N_MIN)

    def body(i, t):
        cand = t + jax.lax.shift_left(jnp.int32(1), 30 - i)
        cnt = jnp.sum((p >= cand).astype(jnp.int32))
        return jnp.where(cnt >= k, cand, t)

    t = jax.lax.fori_loop(0, 31, body, jnp.int32(0))
    tv = jax.lax.bitcast_convert_type(t, jnp.float32)

    gt = p > t
    cnt_gt = jnp.sum(gt.astype(jnp.int32))
    sum_gt = jnp.sum(jnp.where(gt, v, 0.0))
    out_ref[0, 0] = (sum_gt + (k - cnt_gt).astype(jnp.float32) * tv) / jnp.float32(_N_MIN)


def _topk_branch(logits, labels):
    N, C, H, W = logits.shape
    BH = 256
    loss = pl.pallas_call(
        _ce_block,
        grid=(N, H // BH),
        in_specs=[
            pl.BlockSpec((1, C, BH, W), lambda n, h: (n, 0, h, 0)),
            pl.BlockSpec((1, BH, W), lambda n, h: (n, h, 0)),
        ],
        out_specs=pl.BlockSpec((1, BH, W), lambda n, h: (n, h, 0)),
        out_shape=jax.ShapeDtypeStruct((N, H, W), jnp.float32),
    )(logits, labels)
    flat = loss.reshape(N * H * W // 1024, 1024)
    out = pl.pallas_call(
        _select_block,
        in_specs=[pl.BlockSpec(flat.shape, lambda: (0, 0))],
        out_specs=pl.BlockSpec(memory_space=pltpu.SMEM),
        out_shape=jax.ShapeDtypeStruct((1, 1), jnp.float32),
    )(flat)
    return out[0, 0]


@jax.jit
def kernel(logits, labels):
    N, C, H, W = logits.shape
    labels = labels.astype(jnp.int32)
    BH = 512
    stats = pl.pallas_call(
        _ce_stats_block,
        grid=(N, H // BH),
        in_specs=[
            pl.BlockSpec((1, C, BH, W), lambda n, h: (n, 0, h, 0)),
            pl.BlockSpec((1, BH, W), lambda n, h: (n, h, 0)),
        ],
        out_specs=pl.BlockSpec(memory_space=pltpu.SMEM),
        out_shape=jax.ShapeDtypeStruct((3,), jnp.float32),
        scratch_shapes=[pltpu.VMEM((2, BH, W), jnp.float32)],
    )(logits, labels)

    return jax.lax.cond(
        stats[1] >= jnp.float32(_N_MIN),
        lambda: stats[2],
        lambda: _topk_branch(logits, labels),
    )
